# baseline (speedup 1.0000x reference)
import functools

import jax
import jax.numpy as jnp
import numpy as np
from jax import lax
from jax.experimental import pallas as pl
from jax.experimental.pallas import tpu as pltpu
from jax.experimental.pallas import tpu_sc as plsc

_NODE_TYPES = ['author', 'paper', 'term', 'conf']
_NNODES = {'author': 10000, 'paper': 20000, 'term': 4000, 'conf': 20}
_EDGE_TYPES = [('author', 'paper'), ('paper', 'author'), ('paper', 'term'),
               ('term', 'paper'), ('paper', 'conf'), ('conf', 'paper')]
_HID = 256
_HEADS = 8
_DH = _HID // _HEADS
_NLAYERS = 2
_R = len(_EDGE_TYPES)

_EB = 64
_BN = 512
_CHT = {'author': 10240, 'paper': 10240, 'term': 5120, 'conf': 2048}


def _erf(x):
    s = jnp.sign(x)
    ax = jnp.abs(x)
    t = 1.0 / (1.0 + 0.3275911 * ax)
    poly = t * (0.254829592 + t * (-0.284496736 + t * (1.421413741
               + t * (-1.453152027 + t * 1.061405429))))
    return s * (1.0 - poly * jnp.exp(-ax * ax))


def _gelu(x):
    return 0.5 * x * (1.0 + _erf(x * np.float32(1.0 / np.sqrt(2.0))))


def _mm_body(x_ref, w_ref, b_ref, o_ref, *, act):
    y = jnp.dot(x_ref[...], w_ref[...], preferred_element_type=jnp.float32)
    y = y + b_ref[...]
    if act == 'relu':
        y = jnp.maximum(y, 0.0)
    o_ref[...] = y


def _mm(x, w, b, act=None, bn=_BN):
    n, k = x.shape
    m = w.shape[1]
    npad = -(-n // bn) * bn
    if npad != n:
        x = jnp.pad(x, ((0, npad - n), (0, 0)))
    out = pl.pallas_call(
        functools.partial(_mm_body, act=act),
        grid=(npad // bn,),
        in_specs=[
            pl.BlockSpec((bn, k), lambda i: (i, 0)),
            pl.BlockSpec((k, m), lambda i: (0, 0)),
            pl.BlockSpec((1, m), lambda i: (0, 0)),
        ],
        out_specs=pl.BlockSpec((bn, m), lambda i: (i, 0)),
        out_shape=jax.ShapeDtypeStruct((npad, m), jnp.float32),
    )(x, w, b.reshape(1, m))
    return out[:n] if npad != n else out


def _mms_body(x_ref, w_ref, b_ref, *o_refs):
    y = jnp.dot(x_ref[...], w_ref[0], preferred_element_type=jnp.float32)
    y = y + b_ref[0]
    for g, o in enumerate(o_refs):
        o[0] = y[:, g * 128:(g + 1) * 128]


def _mm_multi(x, w, b, bn=_BN):
    n, kdim = x.shape
    m = w.shape[1]
    G = m // 256
    w2 = w.reshape(kdim, G, 2, 128).transpose(2, 0, 1, 3).reshape(2, kdim, G * 128)
    b2 = b.reshape(G, 2, 128).swapaxes(0, 1).reshape(2, 1, G * 128)
    npad = -(-n // bn) * bn
    if npad != n:
        x = jnp.pad(x, ((0, npad - n), (0, 0)))
    outs = pl.pallas_call(
        _mms_body,
        grid=(2, npad // bn),
        in_specs=[
            pl.BlockSpec((bn, kdim), lambda c, i: (i, 0)),
            pl.BlockSpec((1, kdim, G * 128), lambda c, i: (c, 0, 0)),
            pl.BlockSpec((1, 1, G * 128), lambda c, i: (c, 0, 0)),
        ],
        out_specs=[pl.BlockSpec((1, bn, 128), lambda c, i: (c, i, 0))
                   for _ in range(G)],
        out_shape=[jax.ShapeDtypeStruct((2, npad, 128), jnp.float32)
                   for _ in range(G)],
    )(x, w2, b2)
    return outs, npad


def _fold_body(lw_ref, lb_ref, rhs_ref, ow_ref, ob_ref):
    ow_ref[0] = jnp.dot(lw_ref[0], rhs_ref[0],
                        preferred_element_type=jnp.float32)
    ob_ref[0] = jnp.dot(lb_ref[0], rhs_ref[0],
                        preferred_element_type=jnp.float32)


def _fold(lw, lb, rhs):
    B = lw.shape[0]
    return pl.pallas_call(
        _fold_body,
        grid=(B,),
        in_specs=[
            pl.BlockSpec((1, _HID, _HID), lambda i: (i, 0, 0)),
            pl.BlockSpec((1, 1, _HID), lambda i: (i, 0, 0)),
            pl.BlockSpec((1, _HID, _HID), lambda i: (i, 0, 0)),
        ],
        out_specs=[
            pl.BlockSpec((1, _HID, _HID), lambda i: (i, 0, 0)),
            pl.BlockSpec((1, 1, _HID), lambda i: (i, 0, 0)),
        ],
        out_shape=[
            jax.ShapeDtypeStruct((B, _HID, _HID), jnp.float32),
            jax.ShapeDtypeStruct((B, 1, _HID), jnp.float32),
        ],
    )(lw, lb.reshape(B, 1, _HID), rhs)


def _blockdiag(a):
    eye = jnp.eye(_HEADS, dtype=a.dtype)
    out = eye[:, None, :, None] * a[..., :, :, None, :]
    return out.reshape(a.shape[:-3] + (_HID, _HID))


def _edge_pad(e):
    ept = -(-e // (16 * _EB)) * _EB
    return 16 * ept, ept


def _make_edge_sc(NS, ND, E, d0, ch):
    e_pad, ept = _edge_pad(E)
    nblk = ept // _EB
    SPR = ch + 16
    NG = _EB // 16
    NB = ch // (16 * _EB)
    RT = ch // 16
    RT8 = ch * 8 // 16
    assert ch % (16 * _EB) == 0
    mesh = plsc.VectorSubcoreMesh(core_axis_name="c", subcore_axis_name="s")

    @functools.partial(
        pl.kernel, mesh=mesh,
        compiler_params=pltpu.CompilerParams(needs_layout_passes=False, use_tc_tiling_on_sc=False),
        out_type=[jax.ShapeDtypeStruct((2 * ch * 8, 16), jnp.float32),
                  jax.ShapeDtypeStruct((2 * ch, 16), jnp.float32)],
        scratch_types=[
            pltpu.VMEM((_EB,), jnp.int32),
            pltpu.VMEM((_EB,), jnp.int32),
            pltpu.VMEM((_EB,), jnp.int32),
            pltpu.VMEM((_EB,), jnp.int32),
            pltpu.VMEM((_EB,), jnp.int32),
            pltpu.VMEM((8 * _EB,), jnp.int32),
            pltpu.VMEM((_EB, 128), jnp.float32),
            pltpu.VMEM((_EB, 128), jnp.float32),
            pltpu.VMEM((_EB, 16), jnp.float32),
            pltpu.VMEM((16, 16), jnp.float32),
            pltpu.VMEM((8 * _EB, 16), jnp.float32),
            pltpu.VMEM_SHARED((SPR * 8, 16), jnp.float32),
            pltpu.VMEM_SHARED((SPR, 16), jnp.float32),
            pltpu.SemaphoreType.DMA,
            pltpu.SemaphoreType.DMA,
        ])
    def body(kt_hbm, vt_hbm, q_hbm, src_hbm, dst_hbm,
             num_hbm, den_hbm,
             srcb, dstb, idxk, idxq, idxn, idxn8, gb1, gb2, wrow, wT, vb8,
             num_sp8, den_sp, sem1, sem2):
        cid = lax.axis_index("c")
        sid = lax.axis_index("s")
        lane = lax.iota(jnp.int32, 16)
        zv = jnp.zeros((16,), jnp.float32)

        for e in range(_EB):
            wrow[e, :] = zv
        for r in range(8 * _EB):
            vb8[r, :] = zv
        for j in range(NB):
            for i in range(8 * _EB // 16):
                idxn8[pl.ds(i * 16, 16)] = (lane + sid * RT8
                                            + j * 8 * _EB + i * 16)
            pltpu.sync_copy(vb8, num_sp8.at[idxn8])
        for j in range(NB):
            for i in range(NG):
                idxn[pl.ds(i * 16, 16)] = (lane + sid * RT
                                           + j * _EB + i * 16)
            pltpu.sync_copy(wrow, den_sp.at[idxn])
        plsc.subcore_barrier()

        def blk_body(b, _):
            off = sid * ept + b * _EB
            pltpu.sync_copy(src_hbm.at[pl.ds(off, _EB)], srcb)
            pltpu.sync_copy(dst_hbm.at[pl.ds(off, _EB)], dstb)
            for i in range(NG):
                sl = pl.ds(i * 16, 16)
                sv = srcb[sl]
                dv = dstb[sl]
                idxk[sl] = sv + cid * NS
                idxq[sl] = jnp.minimum(dv, ND - 1) + cid * ND
                inb = (dv >= d0) & (dv < d0 + ch)
                idxn[sl] = jnp.where(inb, dv - d0, ch)
            ck = pltpu.async_copy(kt_hbm.at[idxk], gb1, sem1)
            cq = pltpu.async_copy(q_hbm.at[idxq], gb2, sem2)
            ck.wait()
            cq.wait()

            for g in range(NG):
                rows = lane + g * 16
                for h in range(4):
                    def a_body(s, acc, h=h, rows=rows):
                        for u in range(4):
                            col = lane * 0 + (h * 32 + s * 4 + u)
                            kc = plsc.load_gather(gb1, [rows, col])
                            qc = plsc.load_gather(gb2, [rows, col])
                            acc = acc + kc * qc
                        return acc
                    acc_h = lax.fori_loop(0, 8, a_body,
                                          jnp.zeros((16,), jnp.float32))
                    wT[g * 4 + h, :] = jnp.exp(acc_h)

            for g in range(NG):
                ir = g * 4 + jnp.minimum(lane, 3)
                for j in range(16):
                    val = plsc.load_gather(wT, [ir, lane * 0 + j])
                    wrow[g * 16 + j, :] = jnp.where(lane < 4, val, 0.0)

            for i in range(8 * _EB // 16):
                ev = i * 2 + (lane >= 8).astype(jnp.int32)
                base = plsc.load_gather(idxn, [ev])
                idxn8[pl.ds(i * 16, 16)] = base * 8 + (lane & 7)

            cv = pltpu.async_copy(vt_hbm.at[idxk], gb1, sem1)
            cv.wait()

            def sc_body(e, carry):
                er = lane * 0 + e
                for p in range(8):
                    vc = plsc.load_gather(gb1, [er, lane + p * 16])
                    bc = plsc.load_gather(wrow, [er, lane * 0 + (p // 2)])
                    vb8[e * 8 + p, :] = vc * bc
                return carry
            lax.fori_loop(0, _EB, sc_body, 0)

            pltpu.sync_copy(vb8, num_sp8.at[idxn8], add=True)
            pltpu.sync_copy(wrow, den_sp.at[idxn], add=True)
            return _

        lax.fori_loop(0, nblk, blk_body, 0)
        plsc.subcore_barrier()
        for j in range(NB):
            for i in range(8 * _EB // 16):
                idxn8[pl.ds(i * 16, 16)] = (lane + sid * RT8
                                            + j * 8 * _EB + i * 16)
            pltpu.async_copy(num_sp8.at[idxn8], vb8, sem1).wait()
            pltpu.sync_copy(vb8, num_hbm.at[pl.ds(cid * ch * 8 + sid * RT8
                                                  + j * 8 * _EB, 8 * _EB)])
        for j in range(NB):
            for i in range(NG):
                idxn[pl.ds(i * 16, 16)] = (lane + sid * RT
                                           + j * _EB + i * 16)
            pltpu.async_copy(den_sp.at[idxn], wrow, sem2).wait()
            pltpu.sync_copy(wrow, den_hbm.at[pl.ds(cid * ch + sid * RT
                                                   + j * _EB, _EB)])
        plsc.subcore_barrier()

    return body


def _upd_body(a_ref, *refs, nrel):
    nums = refs[:nrel]
    dens = refs[nrel:2 * nrel]
    x_ref, wa_ref, ba_ref, o_ref = refs[2 * nrel:]
    row16 = lax.broadcasted_iota(jnp.int32, (16, 128), 0)
    col16 = lax.broadcasted_iota(jnp.int32, (16, 128), 1)
    e16 = (col16 // _DH == row16).astype(jnp.float32)
    halves = []
    for c in range(2):
        aggc = jnp.zeros((x_ref.shape[0], 128), jnp.float32)
        for j in range(nrel):
            num = nums[j][c]
            den = dens[j][c]
            denb = jnp.dot(den, e16, preferred_element_type=jnp.float32)
            recip = jnp.where(denb > 0, 1.0 / jnp.maximum(denb, 1e-30), 0.0)
            aggc = aggc + num * recip
        halves.append(aggc)
    agg = jnp.concatenate(halves, axis=1)
    o = jnp.dot(_gelu(agg), wa_ref[...],
                preferred_element_type=jnp.float32) + ba_ref[...]
    a = a_ref[0]
    o_ref[...] = a * o + (1.0 - a) * x_ref[...]


def _update(numdens, x, wa, ba, a_gate, bn=_BN):
    n = x.shape[0]
    nrel = len(numdens)
    npad = -(-n // bn) * bn
    if npad != n:
        x = jnp.pad(x, ((0, npad - n), (0, 0)))
    nums = [nu for (nu, _) in numdens]
    dens = [de for (_, de) in numdens]
    out = pl.pallas_call(
        functools.partial(_upd_body, nrel=nrel),
        grid=(npad // bn,),
        in_specs=(
            [pl.BlockSpec(memory_space=pltpu.SMEM)]
            + [pl.BlockSpec((2, bn, 128), lambda i: (0, i, 0))
               for _ in range(nrel)]
            + [pl.BlockSpec((2, bn, 16), lambda i: (0, i, 0))
               for _ in range(nrel)]
            + [pl.BlockSpec((bn, _HID), lambda i: (i, 0)),
               pl.BlockSpec((_HID, _HID), lambda i: (0, 0)),
               pl.BlockSpec((1, _HID), lambda i: (0, 0))]),
        out_specs=pl.BlockSpec((bn, _HID), lambda i: (i, 0)),
        out_shape=jax.ShapeDtypeStruct((npad, _HID), jnp.float32),
    )(a_gate.reshape(1), *nums, *dens, x, wa, ba.reshape(1, _HID))
    return out[:n] if npad != n else out


def kernel(x_author, x_paper, x_term, x_conf, ei_ap, ei_pa, ei_pt, ei_tp,
           ei_pc, ei_cp, Win, bin_, Wk, bk, Wq, bq, Wv, bv, Wa, ba, skip,
           a_rel, m_rel, p_rel, Wout, bout):
    xs = {'author': x_author, 'paper': x_paper, 'term': x_term, 'conf': x_conf}
    eis = [ei_ap, ei_pa, ei_pt, ei_tp, ei_pc, ei_cp]
    ti = {t: i for i, t in enumerate(_NODE_TYPES)}
    nchunks = {t: -(-_NNODES[t] // _CHT[t]) for t in _NODE_TYPES}

    pscale = (p_rel / np.sqrt(_DH))[..., None, None]
    bd_a = _blockdiag(a_rel * pscale)
    bd_m = _blockdiag(m_rel)
    lw, lb, rhs = [], [], []
    for l in range(_NLAYERS):
        for r, (st, dt) in enumerate(_EDGE_TYPES):
            lw += [Wk[l, ti[st]], Wv[l, ti[st]]]
            lb += [bk[l, ti[st]], bv[l, ti[st]]]
            rhs += [bd_a[l, r], bd_m[l, r]]
    wf, bf = _fold(jnp.stack(lw), jnp.stack(lb), jnp.stack(rhs))
    wf_kt = {(l, r): wf[(l * _R + r) * 2] for l in range(_NLAYERS) for r in range(_R)}
    wf_vt = {(l, r): wf[(l * _R + r) * 2 + 1] for l in range(_NLAYERS) for r in range(_R)}
    bf_kt = {(l, r): bf[(l * _R + r) * 2, 0] for l in range(_NLAYERS) for r in range(_R)}
    bf_vt = {(l, r): bf[(l * _R + r) * 2 + 1, 0] for l in range(_NLAYERS) for r in range(_R)}

    x = {}
    for i, t in enumerate(_NODE_TYPES):
        x[t] = _mm(xs[t], Win[i], bin_[i], act='relu')

    for l in range(_NLAYERS):
        q2 = {}
        kt2 = {}
        vt2 = {}
        npads = {}
        for t in _NODE_TYPES:
            rels = [r for r, (st, _) in enumerate(_EDGE_TYPES) if st == t]
            wcat = jnp.concatenate(
                [Wq[l, ti[t]]] + [wf_kt[(l, r)] for r in rels]
                + [wf_vt[(l, r)] for r in rels], axis=1)
            bcat = jnp.concatenate(
                [bq[l, ti[t]]] + [bf_kt[(l, r)] for r in rels]
                + [bf_vt[(l, r)] for r in rels], axis=0)
            outs, npads[t] = _mm_multi(x[t], wcat, bcat)
            q2[t] = outs[0]
            for j, r in enumerate(rels):
                kt2[r] = outs[1 + j]
                vt2[r] = outs[1 + len(rels) + j]

        numden = {t: [] for t in _NODE_TYPES}
        for r, (st, dt) in enumerate(_EDGE_TYPES):
            E = eis[r].shape[1]
            e_pad, _ept = _edge_pad(E)
            ch = _CHT[dt]
            NPd = nchunks[dt] * ch
            srcp = jnp.concatenate(
                [eis[r][0], jnp.zeros((e_pad - E,), jnp.int32)])
            dstp = jnp.concatenate(
                [eis[r][1], jnp.full((e_pad - E,), NPd, jnp.int32)])
            nparts, dparts = [], []
            for ci in range(nchunks[dt]):
                fn = _make_edge_sc(npads[st], npads[dt], E, ci * ch, ch)
                num, den = fn(kt2[r].reshape(-1, 128),
                              vt2[r].reshape(-1, 128),
                              q2[dt].reshape(-1, 128), srcp, dstp)
                nparts.append(num.reshape(2, ch, 128))
                dparts.append(den.reshape(2, ch, 16))
            numf = (nparts[0] if len(nparts) == 1
                    else jnp.concatenate(nparts, axis=1))
            denf = (dparts[0] if len(dparts) == 1
                    else jnp.concatenate(dparts, axis=1))
            numden[dt].append((numf, denf))

        newx = {}
        for i, t in enumerate(_NODE_TYPES):
            a_gate = jax.nn.sigmoid(skip[l, i])
            newx[t] = _update(numden[t], x[t], Wa[l, i], ba[l, i], a_gate)
        x = newx

    wout_p = jnp.pad(Wout, ((0, 0), (0, 128 - Wout.shape[1])))
    bout_p = jnp.pad(bout, (0, 128 - bout.shape[0]))
    return _mm(x['author'], wout_p, bout_p)[:, :bout.shape[0]]
